# baseline (speedup 1.0000x reference)
import functools
import jax
import jax.numpy as jnp
from jax import lax
from jax.experimental import pallas as pl
from jax.experimental.pallas import tpu as pltpu
from jax.experimental.pallas import tpu_sc as plsc

N = 10000
E = 320000
D = 128
DH = 64

B = 128
NC = 2
NS = 16
NBP = 2560
TT = NBP // NS
NBUF = 4
NPAD = 10240
RPS = NPAD // NS
PAD_DST = N


HT = TT // 2


def _sc_segment_sum_body(x_hbm, eidx_hbm, out_hbm, acc_sh, x_sh,
                         src_all, dst_all, rows0, rows1, rows2,
                         semr, semr1, semr2, semw, semw1, semw2):
    c = lax.axis_index("c")
    s = lax.axis_index("s")
    base = s * TT

    pltpu.sync_copy(x_hbm.at[c, pl.ds(s * RPS, RPS)],
                    x_sh.at[pl.ds(s * RPS, RPS)])
    pltpu.sync_copy(x_hbm.at[c, pl.ds(s * RPS, RPS)],
                    acc_sh.at[pl.ds(s * RPS, RPS)])
    plsc.subcore_barrier()

    def start_gather(rws, t, sem):
        pltpu.async_copy(x_sh.at[src_all.at[t]], rws, sem)

    def wait_gather(rws, t, sem):
        pltpu.make_async_copy(x_sh.at[src_all.at[t]], rws, sem).wait()

    def start_scatter(rws, t, sem):
        pltpu.async_copy(rws, acc_sh.at[dst_all.at[t]], sem, add=True)

    def wait_scatter(rws, t, sem):
        pltpu.make_async_copy(rws, acc_sh.at[dst_all.at[t]], sem).wait()

    for h in range(2):
        pltpu.sync_copy(eidx_hbm.at[0, pl.ds(base + h * HT, HT)], src_all)
        pltpu.sync_copy(eidx_hbm.at[1, pl.ds(base + h * HT, HT)], dst_all)

        start_gather(rows0, 0, semr)
        start_gather(rows1, 1, semr1)
        start_gather(rows2, 2, semr2)

        def triple(k, carry):
            t0 = 3 * k
            wait_gather(rows0, t0, semr)
            start_scatter(rows0, t0, semw)
            wait_gather(rows1, t0 + 1, semr1)
            start_scatter(rows1, t0 + 1, semw1)
            wait_gather(rows2, t0 + 2, semr2)
            start_scatter(rows2, t0 + 2, semw2)
            wait_scatter(rows0, t0, semw)
            start_gather(rows0, t0 + 3, semr)
            wait_scatter(rows1, t0 + 1, semw1)
            start_gather(rows1, t0 + 4, semr1)
            wait_scatter(rows2, t0 + 2, semw2)
            start_gather(rows2, t0 + 5, semr2)
            return carry

        lax.fori_loop(0, HT // 3 - 1, triple, 0)

        t0 = HT - 5
        wait_gather(rows0, t0, semr)
        start_scatter(rows0, t0, semw)
        wait_gather(rows1, t0 + 1, semr1)
        start_scatter(rows1, t0 + 1, semw1)
        wait_gather(rows2, t0 + 2, semr2)
        start_scatter(rows2, t0 + 2, semw2)
        wait_scatter(rows0, t0, semw)
        start_gather(rows0, t0 + 3, semr)
        wait_scatter(rows1, t0 + 1, semw1)
        start_gather(rows1, t0 + 4, semr1)
        wait_scatter(rows2, t0 + 2, semw2)
        wait_gather(rows0, t0 + 3, semr)
        start_scatter(rows0, t0 + 3, semw)
        wait_gather(rows1, t0 + 4, semr1)
        start_scatter(rows1, t0 + 4, semw1)
        wait_scatter(rows0, t0 + 3, semw)
        wait_scatter(rows1, t0 + 4, semw1)

    plsc.subcore_barrier()
    pltpu.sync_copy(acc_sh.at[pl.ds(s * RPS, RPS)],
                    out_hbm.at[c, pl.ds(s * RPS, RPS)])


@jax.jit
def _sc_segment_sum(x_st, eidx):
    mesh = plsc.VectorSubcoreMesh(core_axis_name="c", subcore_axis_name="s")
    f = pl.kernel(
        _sc_segment_sum_body,
        out_type=jax.ShapeDtypeStruct((NC, NPAD, DH), jnp.float32),
        mesh=mesh,
        compiler_params=pltpu.CompilerParams(use_tc_tiling_on_sc=False),
        scratch_types=(
            [pltpu.VMEM_SHARED((NPAD, DH), jnp.float32)]
            + [pltpu.VMEM_SHARED((NPAD, DH), jnp.float32)]
            + [pltpu.VMEM((HT, B), jnp.int32) for _ in range(2)]
            + [pltpu.VMEM((B, DH), jnp.float32) for _ in range(3)]
            + [pltpu.SemaphoreType.DMA for _ in range(6)]
        ),
    )
    return f(x_st, eidx)


def _tc_dense_mid_body(a_ref, W1_ref, b1_ref, g_ref, be_ref, W2_ref, b2_ref,
                       out_ref):
    h = (
        jnp.dot(a_ref[0, 0:N], W1_ref[0:DH],
                preferred_element_type=jnp.float32)
        + jnp.dot(a_ref[1, 0:N], W1_ref[DH:D],
                  preferred_element_type=jnp.float32)
        + b1_ref[...]
    )
    mu = jnp.mean(h, axis=0, keepdims=True)
    hc = h - mu
    var = jnp.mean(hc * hc, axis=0, keepdims=True)
    h = hc / jnp.sqrt(var + 1e-5) * g_ref[...] + be_ref[...]
    h = jnp.maximum(h, 0.0)
    out_ref[0, 0:N] = (
        jnp.dot(h, W2_ref[:, 0:DH], preferred_element_type=jnp.float32)
        + b2_ref[:, 0:DH]
    )
    out_ref[1, 0:N] = (
        jnp.dot(h, W2_ref[:, DH:D], preferred_element_type=jnp.float32)
        + b2_ref[:, DH:D]
    )


def _tc_dense_fin_body(a_ref, W1_ref, b1_ref, g_ref, be_ref, W2_ref, b2_ref,
                       out_ref):
    h = (
        jnp.dot(a_ref[0, 0:N], W1_ref[0:DH],
                preferred_element_type=jnp.float32)
        + jnp.dot(a_ref[1, 0:N], W1_ref[DH:D],
                  preferred_element_type=jnp.float32)
        + b1_ref[...]
    )
    mu = jnp.mean(h, axis=0, keepdims=True)
    hc = h - mu
    var = jnp.mean(hc * hc, axis=0, keepdims=True)
    h = hc / jnp.sqrt(var + 1e-5) * g_ref[...] + be_ref[...]
    h = jnp.maximum(h, 0.0)
    out_ref[...] = (
        jnp.dot(h, W2_ref[...], preferred_element_type=jnp.float32)
        + b2_ref[...]
    )


@jax.jit
def _tc_dense_mid(a, W1, b1, g, be, W2, b2):
    return pl.pallas_call(
        _tc_dense_mid_body,
        out_shape=jax.ShapeDtypeStruct((NC, NPAD, DH), jnp.float32),
    )(a, W1, b1.reshape(1, D), g.reshape(1, D), be.reshape(1, D),
      W2, b2.reshape(1, D))


@jax.jit
def _tc_dense_fin(a, W1, b1, g, be, W2, b2):
    return pl.pallas_call(
        _tc_dense_fin_body,
        out_shape=jax.ShapeDtypeStruct((N, D), jnp.float32),
    )(a, W1, b1.reshape(1, D), g.reshape(1, D), be.reshape(1, D),
      W2, b2.reshape(1, D))


@jax.jit
def _prep(x, edge_index):
    eidx = edge_index.reshape(2, E // B, B)
    npad = NBP - E // B
    pad = jnp.stack([
        jnp.zeros((npad, B), jnp.int32),
        jnp.full((npad, B), PAD_DST, jnp.int32),
    ])
    eidx = jnp.concatenate([eidx, pad], axis=1)
    x_st = jnp.stack([x[:, 0:DH], x[:, DH:D]])
    x_st = jnp.pad(x_st, ((0, 0), (0, NPAD - N), (0, 0)))
    return x_st, eidx


def kernel(x, edge_index,
           W1_0, b1_0, g_0, be_0, W2_0, b2_0,
           W1_1, b1_1, g_1, be_1, W2_1, b2_1,
           W1_2, b1_2, g_2, be_2, W2_2, b2_2):
    x_st, eidx = _prep(x, edge_index)
    a = _sc_segment_sum(x_st, eidx)
    x_st = _tc_dense_mid(a, W1_0, b1_0, g_0, be_0, W2_0, b2_0)
    a = _sc_segment_sum(x_st, eidx)
    x_st = _tc_dense_mid(a, W1_1, b1_1, g_1, be_1, W2_1, b2_1)
    a = _sc_segment_sum(x_st, eidx)
    return _tc_dense_fin(a, W1_2, b1_2, g_2, be_2, W2_2, b2_2)


# speedup vs baseline: 1.2420x; 1.2420x over previous
import functools
import jax
import jax.numpy as jnp
from jax import lax
from jax.experimental import pallas as pl
from jax.experimental.pallas import tpu as pltpu
from jax.experimental.pallas import tpu_sc as plsc

N = 10000
E = 320000
D = 128
DH = 64

B = 128
NC = 2
NS = 16
NBP = 2560
TT = NBP // NS
NBUF = 4
NPAD = 10240
RPS = NPAD // NS
PAD_DST = N


HT = TT // 2


def _sc_segment_sum_body(x_hbm, eidx_hbm, out_hbm, acc_sh, x_sh,
                         src_all, dst_all, rows0, rows1, rows2, semr, semr1, semr2):
    c = lax.axis_index("c")
    s = lax.axis_index("s")
    base = s * TT

    pltpu.async_copy(x_hbm.at[c, pl.ds(s * RPS, RPS)],
                     x_sh.at[pl.ds(s * RPS, RPS)], semr)
    pltpu.async_copy(x_hbm.at[c, pl.ds(s * RPS, RPS)],
                     acc_sh.at[pl.ds(s * RPS, RPS)], semr1)
    pltpu.async_copy(eidx_hbm.at[0, pl.ds(base, HT)], src_all, semr2)
    pltpu.async_copy(eidx_hbm.at[1, pl.ds(base, HT)], dst_all, semr2)
    pltpu.make_async_copy(x_hbm.at[c, pl.ds(s * RPS, RPS)],
                          x_sh.at[pl.ds(s * RPS, RPS)], semr).wait()
    pltpu.make_async_copy(x_hbm.at[c, pl.ds(s * RPS, RPS)],
                          acc_sh.at[pl.ds(s * RPS, RPS)], semr1).wait()
    pltpu.make_async_copy(eidx_hbm.at[0, pl.ds(base, HT)], src_all,
                          semr2).wait()
    pltpu.make_async_copy(eidx_hbm.at[1, pl.ds(base, HT)], dst_all,
                          semr2).wait()
    plsc.subcore_barrier()

    def start_gather(rws, t, sem):
        pltpu.async_copy(x_sh.at[src_all.at[t]], rws, sem)

    def wait_gather(rws, t, sem):
        pltpu.make_async_copy(x_sh.at[src_all.at[t]], rws, sem).wait()

    def scatter(rws, t):
        pltpu.sync_copy(rws, acc_sh.at[dst_all.at[t]], add=True)

    for h in range(2):
        if h > 0:
            pltpu.sync_copy(eidx_hbm.at[0, pl.ds(base + h * HT, HT)], src_all)
            pltpu.sync_copy(eidx_hbm.at[1, pl.ds(base + h * HT, HT)], dst_all)

        start_gather(rows0, 0, semr)
        start_gather(rows1, 1, semr1)
        start_gather(rows2, 2, semr2)

        def triple(k, carry):
            t0 = 3 * k
            wait_gather(rows0, t0, semr)
            scatter(rows0, t0)
            start_gather(rows0, t0 + 3, semr)
            wait_gather(rows1, t0 + 1, semr1)
            scatter(rows1, t0 + 1)
            start_gather(rows1, t0 + 4, semr1)
            wait_gather(rows2, t0 + 2, semr2)
            scatter(rows2, t0 + 2)
            start_gather(rows2, t0 + 5, semr2)
            return carry

        lax.fori_loop(0, HT // 3 - 1, triple, 0)

        t0 = HT - 5
        wait_gather(rows0, t0, semr)
        scatter(rows0, t0)
        start_gather(rows0, t0 + 3, semr)
        wait_gather(rows1, t0 + 1, semr1)
        scatter(rows1, t0 + 1)
        start_gather(rows1, t0 + 4, semr1)
        wait_gather(rows2, t0 + 2, semr2)
        scatter(rows2, t0 + 2)
        wait_gather(rows0, t0 + 3, semr)
        scatter(rows0, t0 + 3)
        wait_gather(rows1, t0 + 4, semr1)
        scatter(rows1, t0 + 4)

    plsc.subcore_barrier()
    pltpu.sync_copy(acc_sh.at[pl.ds(s * RPS, RPS)],
                    out_hbm.at[c, pl.ds(s * RPS, RPS)])


@jax.jit
def _sc_segment_sum(x_st, eidx):
    mesh = plsc.VectorSubcoreMesh(core_axis_name="c", subcore_axis_name="s")
    f = pl.kernel(
        _sc_segment_sum_body,
        out_type=jax.ShapeDtypeStruct((NC, NPAD, DH), jnp.float32),
        mesh=mesh,
        compiler_params=pltpu.CompilerParams(use_tc_tiling_on_sc=False),
        scratch_types=(
            [pltpu.VMEM_SHARED((NPAD, DH), jnp.float32)]
            + [pltpu.VMEM_SHARED((NPAD, DH), jnp.float32)]
            + [pltpu.VMEM((HT, B), jnp.int32) for _ in range(2)]
            + [pltpu.VMEM((B, DH), jnp.float32) for _ in range(3)]
            + [pltpu.SemaphoreType.DMA for _ in range(3)]
        ),
    )
    return f(x_st, eidx)


def _tc_dense_mid_body(a_ref, W1_ref, b1_ref, g_ref, be_ref, W2_ref, b2_ref,
                       out_ref):
    h = (
        jnp.dot(a_ref[0, 0:N], W1_ref[0:DH],
                preferred_element_type=jnp.float32)
        + jnp.dot(a_ref[1, 0:N], W1_ref[DH:D],
                  preferred_element_type=jnp.float32)
        + b1_ref[...]
    )
    mu = jnp.mean(h, axis=0, keepdims=True)
    hc = h - mu
    var = jnp.mean(hc * hc, axis=0, keepdims=True)
    h = hc / jnp.sqrt(var + 1e-5) * g_ref[...] + be_ref[...]
    h = jnp.maximum(h, 0.0)
    out_ref[0, 0:N] = (
        jnp.dot(h, W2_ref[:, 0:DH], preferred_element_type=jnp.float32)
        + b2_ref[:, 0:DH]
    )
    out_ref[1, 0:N] = (
        jnp.dot(h, W2_ref[:, DH:D], preferred_element_type=jnp.float32)
        + b2_ref[:, DH:D]
    )


def _tc_dense_fin_body(a_ref, W1_ref, b1_ref, g_ref, be_ref, W2_ref, b2_ref,
                       out_ref):
    h = (
        jnp.dot(a_ref[0, 0:N], W1_ref[0:DH],
                preferred_element_type=jnp.float32)
        + jnp.dot(a_ref[1, 0:N], W1_ref[DH:D],
                  preferred_element_type=jnp.float32)
        + b1_ref[...]
    )
    mu = jnp.mean(h, axis=0, keepdims=True)
    hc = h - mu
    var = jnp.mean(hc * hc, axis=0, keepdims=True)
    h = hc / jnp.sqrt(var + 1e-5) * g_ref[...] + be_ref[...]
    h = jnp.maximum(h, 0.0)
    out_ref[...] = (
        jnp.dot(h, W2_ref[...], preferred_element_type=jnp.float32)
        + b2_ref[...]
    )


@jax.jit
def _tc_dense_mid(a, W1, b1, g, be, W2, b2):
    return pl.pallas_call(
        _tc_dense_mid_body,
        out_shape=jax.ShapeDtypeStruct((NC, NPAD, DH), jnp.float32),
    )(a, W1, b1.reshape(1, D), g.reshape(1, D), be.reshape(1, D),
      W2, b2.reshape(1, D))


@jax.jit
def _tc_dense_fin(a, W1, b1, g, be, W2, b2):
    return pl.pallas_call(
        _tc_dense_fin_body,
        out_shape=jax.ShapeDtypeStruct((N, D), jnp.float32),
    )(a, W1, b1.reshape(1, D), g.reshape(1, D), be.reshape(1, D),
      W2, b2.reshape(1, D))


@jax.jit
def _prep(x, edge_index):
    eidx = edge_index.reshape(2, E // B, B)
    npad = NBP - E // B
    pad = jnp.stack([
        jnp.zeros((npad, B), jnp.int32),
        jnp.full((npad, B), PAD_DST, jnp.int32),
    ])
    eidx = jnp.concatenate([eidx, pad], axis=1)
    x_st = jnp.stack([x[:, 0:DH], x[:, DH:D]])
    x_st = jnp.pad(x_st, ((0, 0), (0, NPAD - N), (0, 0)))
    return x_st, eidx


def kernel(x, edge_index,
           W1_0, b1_0, g_0, be_0, W2_0, b2_0,
           W1_1, b1_1, g_1, be_1, W2_1, b2_1,
           W1_2, b1_2, g_2, be_2, W2_2, b2_2):
    x_st, eidx = _prep(x, edge_index)
    a = _sc_segment_sum(x_st, eidx)
    x_st = _tc_dense_mid(a, W1_0, b1_0, g_0, be_0, W2_0, b2_0)
    a = _sc_segment_sum(x_st, eidx)
    x_st = _tc_dense_mid(a, W1_1, b1_1, g_1, be_1, W2_1, b2_1)
    a = _sc_segment_sum(x_st, eidx)
    return _tc_dense_fin(a, W1_2, b1_2, g_2, be_2, W2_2, b2_2)


# speedup vs baseline: 1.2421x; 1.0001x over previous
import jax
import jax.numpy as jnp
from jax import lax
from jax.experimental import pallas as pl
from jax.experimental.pallas import tpu as pltpu
from jax.experimental.pallas import tpu_sc as plsc

N = 10000
E = 320000
D = 128
DH = 64

B = 128
NC = 2
NS = 16
NBP = 2560
TT = NBP // NS
NPAD = 10240
RPS = NPAD // NS
PAD_DST = N


HT = TT // 2


def _sc_segment_sum_body(x_hbm, eidx_hbm, out_hbm, acc_sh, x_sh,
                         src_all, dst_all, rows0, rows1, rows2, semr, semr1, semr2):
    c = lax.axis_index("c")
    s = lax.axis_index("s")
    base = s * TT

    pltpu.async_copy(x_hbm.at[c, pl.ds(s * RPS, RPS)],
                     x_sh.at[pl.ds(s * RPS, RPS)], semr)
    pltpu.async_copy(x_hbm.at[c, pl.ds(s * RPS, RPS)],
                     acc_sh.at[pl.ds(s * RPS, RPS)], semr1)
    pltpu.async_copy(eidx_hbm.at[0, pl.ds(base, HT)], src_all, semr2)
    pltpu.async_copy(eidx_hbm.at[1, pl.ds(base, HT)], dst_all, semr2)
    pltpu.make_async_copy(x_hbm.at[c, pl.ds(s * RPS, RPS)],
                          x_sh.at[pl.ds(s * RPS, RPS)], semr).wait()
    pltpu.make_async_copy(x_hbm.at[c, pl.ds(s * RPS, RPS)],
                          acc_sh.at[pl.ds(s * RPS, RPS)], semr1).wait()
    pltpu.make_async_copy(eidx_hbm.at[0, pl.ds(base, HT)], src_all,
                          semr2).wait()
    pltpu.make_async_copy(eidx_hbm.at[1, pl.ds(base, HT)], dst_all,
                          semr2).wait()
    plsc.subcore_barrier()

    def start_gather(rws, t, sem):
        pltpu.async_copy(x_sh.at[src_all.at[t]], rws, sem)

    def wait_gather(rws, t, sem):
        pltpu.make_async_copy(x_sh.at[src_all.at[t]], rws, sem).wait()

    def scatter(rws, t):
        pltpu.sync_copy(rws, acc_sh.at[dst_all.at[t]], add=True)

    for h in range(2):
        if h > 0:
            pltpu.sync_copy(eidx_hbm.at[0, pl.ds(base + h * HT, HT)], src_all)
            pltpu.sync_copy(eidx_hbm.at[1, pl.ds(base + h * HT, HT)], dst_all)

        start_gather(rows0, 0, semr)
        start_gather(rows1, 1, semr1)
        start_gather(rows2, 2, semr2)

        def triple(k, carry):
            t0 = 3 * k
            wait_gather(rows0, t0, semr)
            scatter(rows0, t0)
            start_gather(rows0, t0 + 3, semr)
            wait_gather(rows1, t0 + 1, semr1)
            scatter(rows1, t0 + 1)
            start_gather(rows1, t0 + 4, semr1)
            wait_gather(rows2, t0 + 2, semr2)
            scatter(rows2, t0 + 2)
            start_gather(rows2, t0 + 5, semr2)
            return carry

        lax.fori_loop(0, HT // 3 - 1, triple, 0)

        t0 = HT - 5
        wait_gather(rows0, t0, semr)
        scatter(rows0, t0)
        start_gather(rows0, t0 + 3, semr)
        wait_gather(rows1, t0 + 1, semr1)
        scatter(rows1, t0 + 1)
        start_gather(rows1, t0 + 4, semr1)
        wait_gather(rows2, t0 + 2, semr2)
        scatter(rows2, t0 + 2)
        wait_gather(rows0, t0 + 3, semr)
        scatter(rows0, t0 + 3)
        wait_gather(rows1, t0 + 4, semr1)
        scatter(rows1, t0 + 4)

    plsc.subcore_barrier()
    pltpu.sync_copy(acc_sh.at[pl.ds(s * RPS, RPS)],
                    out_hbm.at[c, pl.ds(s * RPS, RPS)])


@jax.jit
def _sc_segment_sum(x_st, eidx):
    mesh = plsc.VectorSubcoreMesh(core_axis_name="c", subcore_axis_name="s")
    f = pl.kernel(
        _sc_segment_sum_body,
        out_type=jax.ShapeDtypeStruct((NC, NPAD, DH), jnp.float32),
        mesh=mesh,
        compiler_params=pltpu.CompilerParams(use_tc_tiling_on_sc=False),
        scratch_types=(
            [pltpu.VMEM_SHARED((NPAD, DH), jnp.float32)]
            + [pltpu.VMEM_SHARED((NPAD, DH), jnp.float32)]
            + [pltpu.VMEM((HT, B), jnp.int32) for _ in range(2)]
            + [pltpu.VMEM((B, DH), jnp.float32) for _ in range(3)]
            + [pltpu.SemaphoreType.DMA for _ in range(3)]
        ),
    )
    return f(x_st, eidx)


def _tc_dense_mid_body(a_ref, W1_ref, b1_ref, g_ref, be_ref, W2_ref, b2_ref,
                       out_ref):
    h = (
        jnp.dot(a_ref[0, 0:N], W1_ref[0:DH],
                preferred_element_type=jnp.float32)
        + jnp.dot(a_ref[1, 0:N], W1_ref[DH:D],
                  preferred_element_type=jnp.float32)
        + b1_ref[...]
    )
    mu = jnp.mean(h, axis=0, keepdims=True)
    hc = h - mu
    var = jnp.mean(hc * hc, axis=0, keepdims=True)
    h = hc / jnp.sqrt(var + 1e-5) * g_ref[...] + be_ref[...]
    h = jnp.maximum(h, 0.0)
    out_ref[0, 0:N] = (
        jnp.dot(h, W2_ref[:, 0:DH], preferred_element_type=jnp.float32)
        + b2_ref[:, 0:DH]
    )
    out_ref[1, 0:N] = (
        jnp.dot(h, W2_ref[:, DH:D], preferred_element_type=jnp.float32)
        + b2_ref[:, DH:D]
    )


def _tc_dense_fin_body(a_ref, W1_ref, b1_ref, g_ref, be_ref, W2_ref, b2_ref,
                       out_ref):
    h = (
        jnp.dot(a_ref[0, 0:N], W1_ref[0:DH],
                preferred_element_type=jnp.float32)
        + jnp.dot(a_ref[1, 0:N], W1_ref[DH:D],
                  preferred_element_type=jnp.float32)
        + b1_ref[...]
    )
    mu = jnp.mean(h, axis=0, keepdims=True)
    hc = h - mu
    var = jnp.mean(hc * hc, axis=0, keepdims=True)
    h = hc / jnp.sqrt(var + 1e-5) * g_ref[...] + be_ref[...]
    h = jnp.maximum(h, 0.0)
    out_ref[...] = (
        jnp.dot(h, W2_ref[...], preferred_element_type=jnp.float32)
        + b2_ref[...]
    )


@jax.jit
def _tc_dense_mid(a, W1, b1, g, be, W2, b2):
    return pl.pallas_call(
        _tc_dense_mid_body,
        out_shape=jax.ShapeDtypeStruct((NC, NPAD, DH), jnp.float32),
    )(a, W1, b1.reshape(1, D), g.reshape(1, D), be.reshape(1, D),
      W2, b2.reshape(1, D))


@jax.jit
def _tc_dense_fin(a, W1, b1, g, be, W2, b2):
    return pl.pallas_call(
        _tc_dense_fin_body,
        out_shape=jax.ShapeDtypeStruct((N, D), jnp.float32),
    )(a, W1, b1.reshape(1, D), g.reshape(1, D), be.reshape(1, D),
      W2, b2.reshape(1, D))


@jax.jit
def _prep(x, edge_index):
    eidx = edge_index.reshape(2, E // B, B)
    npad = NBP - E // B
    pad = jnp.stack([
        jnp.zeros((npad, B), jnp.int32),
        jnp.full((npad, B), PAD_DST, jnp.int32),
    ])
    eidx = jnp.concatenate([eidx, pad], axis=1)
    x_st = jnp.stack([x[:, 0:DH], x[:, DH:D]])
    x_st = jnp.pad(x_st, ((0, 0), (0, NPAD - N), (0, 0)))
    return x_st, eidx


def kernel(x, edge_index,
           W1_0, b1_0, g_0, be_0, W2_0, b2_0,
           W1_1, b1_1, g_1, be_1, W2_1, b2_1,
           W1_2, b1_2, g_2, be_2, W2_2, b2_2):
    x_st, eidx = _prep(x, edge_index)
    a = _sc_segment_sum(x_st, eidx)
    x_st = _tc_dense_mid(a, W1_0, b1_0, g_0, be_0, W2_0, b2_0)
    a = _sc_segment_sum(x_st, eidx)
    x_st = _tc_dense_mid(a, W1_1, b1_1, g_1, be_1, W2_1, b2_1)
    a = _sc_segment_sum(x_st, eidx)
    return _tc_dense_fin(a, W1_2, b1_2, g_2, be_2, W2_2, b2_2)
